# baseline (speedup 1.0000x reference)
import functools

import jax
import jax.numpy as jnp
from jax import lax
from jax.experimental import pallas as pl
from jax.experimental.pallas import tpu as pltpu
from jax.experimental.pallas import tpu_sc as plsc

B, L, D = 1, 2048, 1024
H = 16
K = 1536
NU = D // 2
NH = H // 2
HD = NU // NH
R = min(L - K, L // 2)
RU = L - K
HALF = L // 2
UNM = HALF - R

_ROWS = 1024

_HSC = 4
_NW = 32
_RPW = _HSC * L // _NW
_CHR = 16
_LANES = 16


def _sc_colsum_body(scores_hbm, out_hbm, buf_v, acc_v, sem):
    wid = lax.axis_index("s") * 2 + lax.axis_index("c")
    row0 = wid * _RPW

    def _zero(j, _):
        acc_v[pl.ds(j * _LANES, _LANES)] = jnp.zeros((_LANES,), jnp.float32)
        return 0

    lax.fori_loop(0, L // _LANES, _zero, 0)
    iota16 = lax.iota(jnp.int32, _LANES)

    def _chunk(cidx, _):
        r0 = row0 + cidx * _CHR
        pltpu.async_copy(
            scores_hbm.at[pl.ds(r0 * L, _CHR * L)], buf_v, sem).wait()

        def _cols(j, _):
            base = j * _LANES
            acc = acc_v[pl.ds(base, _LANES)]
            for r in range(_CHR):
                acc = acc + buf_v[pl.ds(r * L + base, _LANES)]
            acc_v[pl.ds(base, _LANES)] = acc
            return 0

        lax.fori_loop(0, L // _LANES, _cols, 0)
        i0 = lax.rem(r0, L)
        corr = jnp.zeros((_LANES,), jnp.float32)
        for r in range(_CHR):
            v = buf_v[pl.ds(r * L + i0, _LANES)]
            corr = corr + jnp.where(iota16 == r, v, 0.0)
        acc_v[pl.ds(i0, _LANES)] = acc_v[pl.ds(i0, _LANES)] - corr
        return 0

    lax.fori_loop(0, _RPW // _CHR, _chunk, 0)
    pltpu.sync_copy(acc_v, out_hbm.at[pl.ds(wid * L, L)])


def _sc_colsum(scores_flat):
    k = pl.kernel(
        _sc_colsum_body,
        mesh=plsc.VectorSubcoreMesh(core_axis_name="c", subcore_axis_name="s"),
        out_type=jax.ShapeDtypeStruct((_NW * L,), jnp.float32),
        scratch_types=[
            pltpu.VMEM((_CHR * L,), jnp.float32),
            pltpu.VMEM((L,), jnp.float32),
            pltpu.SemaphoreType.DMA,
        ],
    )
    return k(scores_flat)


def _importance_kernel(s_ref, out_ref):
    c = pl.program_id(0)
    h = pl.program_id(1)
    blk = s_ref[0]
    delta = (jax.lax.broadcasted_iota(jnp.int32, (_ROWS, L), 1)
             - jax.lax.broadcasted_iota(jnp.int32, (_ROWS, L), 0))
    masked = jnp.where(delta == c * _ROWS, 0.0, blk).astype(jnp.bfloat16)
    ones_row = jnp.ones((1, _ROWS), dtype=jnp.bfloat16)
    contrib = jax.lax.dot_general(
        ones_row, masked, (((1,), (0,)), ((), ())),
        preferred_element_type=jnp.float32)

    @pl.when((c == 0) & (h == 0))
    def _():
        out_ref[...] = contrib

    @pl.when(jnp.logical_not((c == 0) & (h == 0)))
    def _():
        out_ref[...] = out_ref[...] + contrib


def _rank_desc_row(v_col, v_row, n, chunk):
    acc = jnp.zeros((1, n), dtype=jnp.float32)
    idx_row = jax.lax.broadcasted_iota(jnp.int32, (1, n), 1)
    for c0 in range(0, n, chunk):
        vi = jax.lax.slice(v_col, (c0, 0), (c0 + chunk, 1))
        idx_col = jax.lax.broadcasted_iota(jnp.int32, (chunk, 1), 0) + c0
        beats = (vi > v_row) | ((vi == v_row) & (idx_col < idx_row))
        acc = acc + jnp.sum(beats.astype(jnp.float32), axis=0, keepdims=True)
    return acc


def _rank_desc_col(v_col, v_row, n, chunk):
    acc = jnp.zeros((n, 1), dtype=jnp.float32)
    idx_col = jax.lax.broadcasted_iota(jnp.int32, (n, 1), 0)
    for c0 in range(0, n, chunk):
        vi = jax.lax.slice(v_row, (0, c0), (1, c0 + chunk))
        idx_row = jax.lax.broadcasted_iota(jnp.int32, (1, chunk), 1) + c0
        beats = (vi > v_col) | ((vi == v_col) & (idx_row < idx_col))
        acc = acc + jnp.sum(beats.astype(jnp.float32), axis=1, keepdims=True)
    return acc


def _select_mha_kernel(imp_row_ref, imp_col_ref, hs_ref,
                       wq_ref, wk_ref, wv_ref, wo_ref, out_ref):
    imp_row = imp_row_ref[...]
    imp_col = imp_col_ref[...]
    rank = _rank_desc_row(imp_col, imp_row, L, 256)
    maskf = (rank >= float(K)).astype(jnp.float32)
    i_col = jax.lax.broadcasted_iota(jnp.int32, (L, 1), 0)
    pos_parts = []
    for c0 in range(0, L, 256):
        j_row = jax.lax.broadcasted_iota(jnp.int32, (1, 256), 1) + c0
        tri = (i_col < j_row).astype(jnp.float32)
        pos_parts.append(jax.lax.dot_general(
            maskf, tri, (((1,), (0,)), ((), ())),
            preferred_element_type=jnp.float32, precision=jax.lax.Precision.HIGHEST))
    pos = jnp.concatenate(pos_parts, axis=1)
    m_col = jax.lax.broadcasted_iota(jnp.int32, (RU, 1), 0).astype(jnp.float32)
    onehot = jnp.where((pos == m_col) & (maskf == 1.0), 1.0, 0.0)
    unp = jax.lax.dot_general(
        onehot, hs_ref[...], (((1,), (0,)), ((), ())),
        preferred_element_type=jnp.float32, precision=jax.lax.Precision.HIGHEST)
    cls = hs_ref[0:1, :]
    q = jnp.dot(cls, wq_ref[...], preferred_element_type=jnp.float32,
                precision=jax.lax.Precision.HIGHEST)
    kx = jnp.dot(unp, wk_ref[...], preferred_element_type=jnp.float32,
                 precision=jax.lax.Precision.HIGHEST)
    vx = jnp.dot(unp, wv_ref[...], preferred_element_type=jnp.float32,
                 precision=jax.lax.Precision.HIGHEST)
    outs = []
    scale = 1.0 / float(HD) ** 0.5
    for h in range(NH):
        qh = q[:, h * HD:(h + 1) * HD]
        kh = kx[:, h * HD:(h + 1) * HD]
        vh = vx[:, h * HD:(h + 1) * HD]
        att = jax.lax.dot_general(
            qh, kh, (((1,), (1,)), ((), ())),
            preferred_element_type=jnp.float32,
            precision=jax.lax.Precision.HIGHEST) * scale
        att = att - jnp.max(att, axis=1, keepdims=True)
        w = jnp.exp(att)
        w = w / jnp.sum(w, axis=1, keepdims=True)
        outs.append(jnp.dot(w, vh, preferred_element_type=jnp.float32,
                            precision=jax.lax.Precision.HIGHEST))
    cat = jnp.concatenate(outs, axis=1)
    out_ref[...] = jnp.dot(cat, wo_ref[...], preferred_element_type=jnp.float32,
                           precision=jax.lax.Precision.HIGHEST)


_BCH = 256


def _node_stats_kernel(an_ref, bn_ref, nmax_ref, nidx_ref):
    an = an_ref[...]
    bn = bn_ref[...]
    scores = jax.lax.dot_general(
        an.astype(jnp.bfloat16), bn.astype(jnp.bfloat16),
        (((1,), (1,)), ((), ())),
        preferred_element_type=jnp.float32)
    nmax = jnp.max(scores, axis=1, keepdims=True)
    j_row = jax.lax.broadcasted_iota(jnp.int32, (1, HALF), 1)
    nidx = jnp.min(jnp.where(scores == nmax, j_row, HALF), axis=1,
                   keepdims=True)
    nmax_ref[...] = nmax
    nidx_ref[...] = nidx


def _merge_kernel(nmax_col_ref, nmax_row_ref, nidx_ref, a_ref, b_ref,
                  unm_ref, dst_ref):
    rank = _rank_desc_col(
        nmax_col_ref[...], nmax_row_ref[...], HALF, 256)
    nidx = nidx_ref[...]
    a = a_ref[...]
    b = b_ref[...]
    m_row = jax.lax.broadcasted_iota(jnp.int32, (1, UNM), 1).astype(jnp.float32)
    unm_oh_t = jnp.where(rank - float(R) == m_row, 1.0, 0.0)
    unm_ref[...] = jax.lax.dot_general(
        unm_oh_t, a, (((0,), (0,)), ((), ())),
        preferred_element_type=jnp.float32, precision=jax.lax.Precision.HIGHEST)
    d_row = jax.lax.broadcasted_iota(jnp.int32, (1, HALF), 1)
    merge_t = jnp.where((nidx == d_row) & (rank < float(R)), 1.0, 0.0)
    accum = jax.lax.dot_general(
        merge_t, a, (((0,), (0,)), ((), ())),
        preferred_element_type=jnp.float32, precision=jax.lax.Precision.HIGHEST)
    ones_col = jnp.ones((HALF, 1), dtype=jnp.float32)
    counts = jax.lax.dot_general(
        merge_t, ones_col, (((0,), (0,)), ((), ())),
        preferred_element_type=jnp.float32, precision=jax.lax.Precision.HIGHEST) + 1.0
    dst_ref[...] = (b + accum) / counts


def kernel(hidden_states, self_attention_scores, Wq, Wk, Wv, Wo):
    scores = self_attention_scores.reshape(H, L, L)
    imp_tc = pl.pallas_call(
        _importance_kernel,
        grid=(L // _ROWS, H - _HSC),
        in_specs=[pl.BlockSpec((1, _ROWS, L), lambda c, h: (h + _HSC, c, 0))],
        out_specs=pl.BlockSpec((1, L), lambda c, h: (0, 0)),
        out_shape=jax.ShapeDtypeStruct((1, L), jnp.float32),
    )(scores)
    partials = _sc_colsum(self_attention_scores.reshape(H * L * L))
    imp = imp_tc + jnp.sum(partials.reshape(_NW, L), axis=0, keepdims=True)

    hs = hidden_states.reshape(L, D)
    new_tok = pl.pallas_call(
        _select_mha_kernel,
        out_shape=jax.ShapeDtypeStruct((1, D), jnp.float32),
    )(imp, imp.reshape(L, 1), hs, Wq, Wk, Wv, Wo)

    a_t = hs[0::2]
    b_t = hs[1::2]
    an_t = a_t / jnp.linalg.norm(a_t, axis=-1, keepdims=True)
    bn_t = b_t / jnp.linalg.norm(b_t, axis=-1, keepdims=True)
    nmax, nidx = pl.pallas_call(
        _node_stats_kernel,
        grid=(HALF // _BCH,),
        in_specs=[
            pl.BlockSpec((_BCH, D), lambda c: (c, 0)),
            pl.BlockSpec((HALF, D), lambda c: (0, 0)),
        ],
        out_specs=[
            pl.BlockSpec((_BCH, 1), lambda c: (c, 0)),
            pl.BlockSpec((_BCH, 1), lambda c: (c, 0)),
        ],
        out_shape=[
            jax.ShapeDtypeStruct((HALF, 1), jnp.float32),
            jax.ShapeDtypeStruct((HALF, 1), jnp.int32),
        ],
    )(an_t, bn_t)

    unm, dst_m = pl.pallas_call(
        _merge_kernel,
        out_shape=[
            jax.ShapeDtypeStruct((UNM, D), jnp.float32),
            jax.ShapeDtypeStruct((HALF, D), jnp.float32),
        ],
    )(nmax, nmax.reshape(1, HALF), nidx, a_t, b_t)

    cls = hs[0:1, :]
    return jnp.concatenate(
        [cls[None], unm[None], dst_m[None], new_tok[None]], axis=1)


# speedup vs baseline: 1.7409x; 1.7409x over previous
import functools

import jax
import jax.numpy as jnp
from jax import lax
from jax.experimental import pallas as pl
from jax.experimental.pallas import tpu as pltpu
from jax.experimental.pallas import tpu_sc as plsc

B, L, D = 1, 2048, 1024
H = 16
K = 1536
NU = D // 2
NH = H // 2
HD = NU // NH
R = min(L - K, L // 2)
RU = L - K
HALF = L // 2
UNM = HALF - R

_ROWS = 1024

_HSC = 4
_NW = 32
_RPW = _HSC * L // _NW
_CHR = 16
_LANES = 16


def _sc_colsum_body(scores_hbm, out_hbm, buf_v, acc_v, sem):
    wid = lax.axis_index("s") * 2 + lax.axis_index("c")
    row0 = wid * _RPW

    def _zero(j, _):
        acc_v[pl.ds(j * _LANES, _LANES)] = jnp.zeros((_LANES,), jnp.float32)
        return 0

    lax.fori_loop(0, L // _LANES, _zero, 0)
    iota16 = lax.iota(jnp.int32, _LANES)

    def _chunk(cidx, _):
        r0 = row0 + cidx * _CHR
        pltpu.async_copy(
            scores_hbm.at[pl.ds(r0, _CHR)], buf_v, sem).wait()

        def _cols(j, _):
            base = j * _LANES
            acc = acc_v[pl.ds(base, _LANES)]
            for r in range(_CHR):
                acc = acc + buf_v[r, pl.ds(base, _LANES)]
            acc_v[pl.ds(base, _LANES)] = acc
            return 0

        lax.fori_loop(0, L // _LANES, _cols, 0)
        i0 = lax.rem(r0, L)
        corr = jnp.zeros((_LANES,), jnp.float32)
        for r in range(_CHR):
            v = buf_v[r, pl.ds(i0, _LANES)]
            corr = corr + jnp.where(iota16 == r, v, 0.0)
        acc_v[pl.ds(i0, _LANES)] = acc_v[pl.ds(i0, _LANES)] - corr
        return 0

    lax.fori_loop(0, _RPW // _CHR, _chunk, 0)
    pltpu.sync_copy(acc_v, out_hbm.at[wid])


def _sc_colsum(scores_rows):
    k = pl.kernel(
        _sc_colsum_body,
        mesh=plsc.VectorSubcoreMesh(core_axis_name="c", subcore_axis_name="s"),
        out_type=jax.ShapeDtypeStruct((_NW, L), jnp.float32),
        scratch_types=[
            pltpu.VMEM((_CHR, L), jnp.float32),
            pltpu.VMEM((L,), jnp.float32),
            pltpu.SemaphoreType.DMA,
        ],
    )
    return k(scores_rows)


def _importance_kernel(s_ref, out_ref):
    c = pl.program_id(0)
    h = pl.program_id(1)
    blk = s_ref[0]
    delta = (jax.lax.broadcasted_iota(jnp.int32, (_ROWS, L), 1)
             - jax.lax.broadcasted_iota(jnp.int32, (_ROWS, L), 0))
    masked = jnp.where(delta == c * _ROWS, 0.0, blk).astype(jnp.bfloat16)
    ones_row = jnp.ones((1, _ROWS), dtype=jnp.bfloat16)
    contrib = jax.lax.dot_general(
        ones_row, masked, (((1,), (0,)), ((), ())),
        preferred_element_type=jnp.float32)

    @pl.when((c == 0) & (h == 0))
    def _():
        out_ref[...] = contrib

    @pl.when(jnp.logical_not((c == 0) & (h == 0)))
    def _():
        out_ref[...] = out_ref[...] + contrib


def _rank_desc_row(v_col, v_row, n, chunk):
    acc = jnp.zeros((1, n), dtype=jnp.float32)
    idx_row = jax.lax.broadcasted_iota(jnp.int32, (1, n), 1)
    for c0 in range(0, n, chunk):
        vi = jax.lax.slice(v_col, (c0, 0), (c0 + chunk, 1))
        idx_col = jax.lax.broadcasted_iota(jnp.int32, (chunk, 1), 0) + c0
        beats = (vi > v_row) | ((vi == v_row) & (idx_col < idx_row))
        acc = acc + jnp.sum(beats.astype(jnp.float32), axis=0, keepdims=True)
    return acc


def _rank_desc_col(v_col, v_row, n, chunk):
    acc = jnp.zeros((n, 1), dtype=jnp.float32)
    idx_col = jax.lax.broadcasted_iota(jnp.int32, (n, 1), 0)
    for c0 in range(0, n, chunk):
        vi = jax.lax.slice(v_row, (0, c0), (1, c0 + chunk))
        idx_row = jax.lax.broadcasted_iota(jnp.int32, (1, chunk), 1) + c0
        beats = (vi > v_col) | ((vi == v_col) & (idx_row < idx_col))
        acc = acc + jnp.sum(beats.astype(jnp.float32), axis=1, keepdims=True)
    return acc


def _select_mha_kernel(imp_row_ref, imp_col_ref, hs_ref,
                       wq_ref, wk_ref, wv_ref, wo_ref, out_ref):
    imp_row = imp_row_ref[...]
    imp_col = imp_col_ref[...]
    rank = _rank_desc_row(imp_col, imp_row, L, 256)
    maskf = (rank >= float(K)).astype(jnp.float32)
    i_col = jax.lax.broadcasted_iota(jnp.int32, (L, 1), 0)
    pos_parts = []
    for c0 in range(0, L, 256):
        j_row = jax.lax.broadcasted_iota(jnp.int32, (1, 256), 1) + c0
        tri = (i_col < j_row).astype(jnp.float32)
        pos_parts.append(jax.lax.dot_general(
            maskf, tri, (((1,), (0,)), ((), ())),
            preferred_element_type=jnp.float32, precision=jax.lax.Precision.HIGHEST))
    pos = jnp.concatenate(pos_parts, axis=1)
    m_col = jax.lax.broadcasted_iota(jnp.int32, (RU, 1), 0).astype(jnp.float32)
    onehot = jnp.where((pos == m_col) & (maskf == 1.0), 1.0, 0.0)
    unp = jax.lax.dot_general(
        onehot, hs_ref[...], (((1,), (0,)), ((), ())),
        preferred_element_type=jnp.float32, precision=jax.lax.Precision.HIGHEST)
    cls = hs_ref[0:1, :]
    q = jnp.dot(cls, wq_ref[...], preferred_element_type=jnp.float32,
                precision=jax.lax.Precision.HIGHEST)
    kx = jnp.dot(unp, wk_ref[...], preferred_element_type=jnp.float32,
                 precision=jax.lax.Precision.HIGHEST)
    vx = jnp.dot(unp, wv_ref[...], preferred_element_type=jnp.float32,
                 precision=jax.lax.Precision.HIGHEST)
    outs = []
    scale = 1.0 / float(HD) ** 0.5
    for h in range(NH):
        qh = q[:, h * HD:(h + 1) * HD]
        kh = kx[:, h * HD:(h + 1) * HD]
        vh = vx[:, h * HD:(h + 1) * HD]
        att = jax.lax.dot_general(
            qh, kh, (((1,), (1,)), ((), ())),
            preferred_element_type=jnp.float32,
            precision=jax.lax.Precision.HIGHEST) * scale
        att = att - jnp.max(att, axis=1, keepdims=True)
        w = jnp.exp(att)
        w = w / jnp.sum(w, axis=1, keepdims=True)
        outs.append(jnp.dot(w, vh, preferred_element_type=jnp.float32,
                            precision=jax.lax.Precision.HIGHEST))
    cat = jnp.concatenate(outs, axis=1)
    out_ref[...] = jnp.dot(cat, wo_ref[...], preferred_element_type=jnp.float32,
                           precision=jax.lax.Precision.HIGHEST)


_BCH = 256


def _node_stats_kernel(an_ref, bn_ref, nmax_ref, nidx_ref):
    an = an_ref[...]
    bn = bn_ref[...]
    scores = jax.lax.dot_general(
        an.astype(jnp.bfloat16), bn.astype(jnp.bfloat16),
        (((1,), (1,)), ((), ())),
        preferred_element_type=jnp.float32)
    nmax = jnp.max(scores, axis=1, keepdims=True)
    j_row = jax.lax.broadcasted_iota(jnp.int32, (1, HALF), 1)
    nidx = jnp.min(jnp.where(scores == nmax, j_row, HALF), axis=1,
                   keepdims=True)
    nmax_ref[...] = nmax
    nidx_ref[...] = nidx


def _merge_kernel(nmax_col_ref, nmax_row_ref, nidx_ref, a_ref, b_ref,
                  unm_ref, dst_ref):
    rank = _rank_desc_col(
        nmax_col_ref[...], nmax_row_ref[...], HALF, 256)
    nidx = nidx_ref[...]
    a = a_ref[...]
    b = b_ref[...]
    m_row = jax.lax.broadcasted_iota(jnp.int32, (1, UNM), 1).astype(jnp.float32)
    unm_oh_t = jnp.where(rank - float(R) == m_row, 1.0, 0.0)
    unm_ref[...] = jax.lax.dot_general(
        unm_oh_t, a, (((0,), (0,)), ((), ())),
        preferred_element_type=jnp.float32, precision=jax.lax.Precision.HIGHEST)
    d_row = jax.lax.broadcasted_iota(jnp.int32, (1, HALF), 1)
    merge_t = jnp.where((nidx == d_row) & (rank < float(R)), 1.0, 0.0)
    accum = jax.lax.dot_general(
        merge_t, a, (((0,), (0,)), ((), ())),
        preferred_element_type=jnp.float32, precision=jax.lax.Precision.HIGHEST)
    ones_col = jnp.ones((HALF, 1), dtype=jnp.float32)
    counts = jax.lax.dot_general(
        merge_t, ones_col, (((0,), (0,)), ((), ())),
        preferred_element_type=jnp.float32, precision=jax.lax.Precision.HIGHEST) + 1.0
    dst_ref[...] = (b + accum) / counts


def kernel(hidden_states, self_attention_scores, Wq, Wk, Wv, Wo):
    scores = self_attention_scores.reshape(H, L, L)
    imp_tc = pl.pallas_call(
        _importance_kernel,
        grid=(L // _ROWS, H - _HSC),
        in_specs=[pl.BlockSpec((1, _ROWS, L), lambda c, h: (h + _HSC, c, 0))],
        out_specs=pl.BlockSpec((1, L), lambda c, h: (0, 0)),
        out_shape=jax.ShapeDtypeStruct((1, L), jnp.float32),
    )(scores)
    partials = _sc_colsum(self_attention_scores.reshape(H * L, L))
    imp = imp_tc + jnp.sum(partials, axis=0, keepdims=True)

    hs = hidden_states.reshape(L, D)
    new_tok = pl.pallas_call(
        _select_mha_kernel,
        out_shape=jax.ShapeDtypeStruct((1, D), jnp.float32),
    )(imp, imp.reshape(L, 1), hs, Wq, Wk, Wv, Wo)

    a_t = hs[0::2]
    b_t = hs[1::2]
    an_t = a_t / jnp.linalg.norm(a_t, axis=-1, keepdims=True)
    bn_t = b_t / jnp.linalg.norm(b_t, axis=-1, keepdims=True)
    nmax, nidx = pl.pallas_call(
        _node_stats_kernel,
        grid=(HALF // _BCH,),
        in_specs=[
            pl.BlockSpec((_BCH, D), lambda c: (c, 0)),
            pl.BlockSpec((HALF, D), lambda c: (0, 0)),
        ],
        out_specs=[
            pl.BlockSpec((_BCH, 1), lambda c: (c, 0)),
            pl.BlockSpec((_BCH, 1), lambda c: (c, 0)),
        ],
        out_shape=[
            jax.ShapeDtypeStruct((HALF, 1), jnp.float32),
            jax.ShapeDtypeStruct((HALF, 1), jnp.int32),
        ],
    )(an_t, bn_t)

    unm, dst_m = pl.pallas_call(
        _merge_kernel,
        out_shape=[
            jax.ShapeDtypeStruct((UNM, D), jnp.float32),
            jax.ShapeDtypeStruct((HALF, D), jnp.float32),
        ],
    )(nmax, nmax.reshape(1, HALF), nidx, a_t, b_t)

    cls = hs[0:1, :]
    return jnp.concatenate(
        [cls[None], unm[None], dst_m[None], new_tok[None]], axis=1)


# speedup vs baseline: 1.7725x; 1.0182x over previous
import functools

import jax
import jax.numpy as jnp
from jax import lax
from jax.experimental import pallas as pl
from jax.experimental.pallas import tpu as pltpu
from jax.experimental.pallas import tpu_sc as plsc

B, L, D = 1, 2048, 1024
H = 16
K = 1536
NU = D // 2
NH = H // 2
HD = NU // NH
R = min(L - K, L // 2)
RU = L - K
HALF = L // 2
UNM = HALF - R

_ROWS = 1024

_HSC = 6
_NW = 32
_RPW = _HSC * L // _NW
_CHR = 16
_LANES = 16


def _sc_colsum_body(scores_hbm, out_hbm, buf_v, acc_v, sem):
    wid = lax.axis_index("s") * 2 + lax.axis_index("c")
    row0 = wid * _RPW

    def _zero(j, _):
        acc_v[pl.ds(j * _LANES, _LANES)] = jnp.zeros((_LANES,), jnp.float32)
        return 0

    lax.fori_loop(0, L // _LANES, _zero, 0)
    iota16 = lax.iota(jnp.int32, _LANES)

    def _chunk(cidx, _):
        r0 = row0 + cidx * _CHR
        pltpu.async_copy(
            scores_hbm.at[pl.ds(r0, _CHR)], buf_v, sem).wait()

        def _cols(j, _):
            base = j * _LANES
            acc = acc_v[pl.ds(base, _LANES)]
            for r in range(_CHR):
                acc = acc + buf_v[r, pl.ds(base, _LANES)]
            acc_v[pl.ds(base, _LANES)] = acc
            return 0

        lax.fori_loop(0, L // _LANES, _cols, 0)
        i0 = lax.rem(r0, L)
        corr = jnp.zeros((_LANES,), jnp.float32)
        for r in range(_CHR):
            v = buf_v[r, pl.ds(i0, _LANES)]
            corr = corr + jnp.where(iota16 == r, v, 0.0)
        acc_v[pl.ds(i0, _LANES)] = acc_v[pl.ds(i0, _LANES)] - corr
        return 0

    lax.fori_loop(0, _RPW // _CHR, _chunk, 0)
    pltpu.sync_copy(acc_v, out_hbm.at[wid])


def _sc_colsum(scores_rows):
    k = pl.kernel(
        _sc_colsum_body,
        mesh=plsc.VectorSubcoreMesh(core_axis_name="c", subcore_axis_name="s"),
        out_type=jax.ShapeDtypeStruct((_NW, L), jnp.float32),
        scratch_types=[
            pltpu.VMEM((_CHR, L), jnp.float32),
            pltpu.VMEM((L,), jnp.float32),
            pltpu.SemaphoreType.DMA,
        ],
    )
    return k(scores_rows)


def _importance_kernel(s_ref, out_ref):
    c = pl.program_id(0)
    h = pl.program_id(1)
    blk = s_ref[0]
    delta = (jax.lax.broadcasted_iota(jnp.int32, (_ROWS, L), 1)
             - jax.lax.broadcasted_iota(jnp.int32, (_ROWS, L), 0))
    masked = jnp.where(delta == c * _ROWS, 0.0, blk).astype(jnp.bfloat16)
    ones_row = jnp.ones((1, _ROWS), dtype=jnp.bfloat16)
    contrib = jax.lax.dot_general(
        ones_row, masked, (((1,), (0,)), ((), ())),
        preferred_element_type=jnp.float32)

    @pl.when((c == 0) & (h == 0))
    def _():
        out_ref[...] = contrib

    @pl.when(jnp.logical_not((c == 0) & (h == 0)))
    def _():
        out_ref[...] = out_ref[...] + contrib


def _rank_desc_row(v_col, v_row, n, chunk):
    acc = jnp.zeros((1, n), dtype=jnp.float32)
    idx_row = jax.lax.broadcasted_iota(jnp.int32, (1, n), 1)
    for c0 in range(0, n, chunk):
        vi = jax.lax.slice(v_col, (c0, 0), (c0 + chunk, 1))
        idx_col = jax.lax.broadcasted_iota(jnp.int32, (chunk, 1), 0) + c0
        beats = (vi > v_row) | ((vi == v_row) & (idx_col < idx_row))
        acc = acc + jnp.sum(beats.astype(jnp.float32), axis=0, keepdims=True)
    return acc


def _rank_desc_col(v_col, v_row, n, chunk):
    acc = jnp.zeros((n, 1), dtype=jnp.float32)
    idx_col = jax.lax.broadcasted_iota(jnp.int32, (n, 1), 0)
    for c0 in range(0, n, chunk):
        vi = jax.lax.slice(v_row, (0, c0), (1, c0 + chunk))
        idx_row = jax.lax.broadcasted_iota(jnp.int32, (1, chunk), 1) + c0
        beats = (vi > v_col) | ((vi == v_col) & (idx_row < idx_col))
        acc = acc + jnp.sum(beats.astype(jnp.float32), axis=1, keepdims=True)
    return acc


def _select_mha_kernel(imp_row_ref, imp_col_ref, hs_ref,
                       wq_ref, wk_ref, wv_ref, wo_ref, out_ref):
    imp_row = imp_row_ref[...]
    imp_col = imp_col_ref[...]
    rank = _rank_desc_row(imp_col, imp_row, L, 256)
    maskf = (rank >= float(K)).astype(jnp.float32)
    i_col = jax.lax.broadcasted_iota(jnp.int32, (L, 1), 0)
    pos_parts = []
    for c0 in range(0, L, 256):
        j_row = jax.lax.broadcasted_iota(jnp.int32, (1, 256), 1) + c0
        tri = (i_col < j_row).astype(jnp.float32)
        pos_parts.append(jax.lax.dot_general(
            maskf, tri, (((1,), (0,)), ((), ())),
            preferred_element_type=jnp.float32, precision=jax.lax.Precision.HIGHEST))
    pos = jnp.concatenate(pos_parts, axis=1)
    m_col = jax.lax.broadcasted_iota(jnp.int32, (RU, 1), 0).astype(jnp.float32)
    onehot = jnp.where((pos == m_col) & (maskf == 1.0), 1.0, 0.0)
    unp = jax.lax.dot_general(
        onehot, hs_ref[...], (((1,), (0,)), ((), ())),
        preferred_element_type=jnp.float32, precision=jax.lax.Precision.HIGHEST)
    cls = hs_ref[0:1, :]
    q = jnp.dot(cls, wq_ref[...], preferred_element_type=jnp.float32,
                precision=jax.lax.Precision.HIGHEST)
    kx = jnp.dot(unp, wk_ref[...], preferred_element_type=jnp.float32,
                 precision=jax.lax.Precision.HIGHEST)
    vx = jnp.dot(unp, wv_ref[...], preferred_element_type=jnp.float32,
                 precision=jax.lax.Precision.HIGHEST)
    outs = []
    scale = 1.0 / float(HD) ** 0.5
    for h in range(NH):
        qh = q[:, h * HD:(h + 1) * HD]
        kh = kx[:, h * HD:(h + 1) * HD]
        vh = vx[:, h * HD:(h + 1) * HD]
        att = jax.lax.dot_general(
            qh, kh, (((1,), (1,)), ((), ())),
            preferred_element_type=jnp.float32,
            precision=jax.lax.Precision.HIGHEST) * scale
        att = att - jnp.max(att, axis=1, keepdims=True)
        w = jnp.exp(att)
        w = w / jnp.sum(w, axis=1, keepdims=True)
        outs.append(jnp.dot(w, vh, preferred_element_type=jnp.float32,
                            precision=jax.lax.Precision.HIGHEST))
    cat = jnp.concatenate(outs, axis=1)
    out_ref[...] = jnp.dot(cat, wo_ref[...], preferred_element_type=jnp.float32,
                           precision=jax.lax.Precision.HIGHEST)


_BCH = 256


def _node_stats_kernel(an_ref, bn_ref, nmax_ref, nidx_ref):
    an = an_ref[...]
    bn = bn_ref[...]
    scores = jax.lax.dot_general(
        an.astype(jnp.bfloat16), bn.astype(jnp.bfloat16),
        (((1,), (1,)), ((), ())),
        preferred_element_type=jnp.float32)
    nmax = jnp.max(scores, axis=1, keepdims=True)
    j_row = jax.lax.broadcasted_iota(jnp.int32, (1, HALF), 1)
    nidx = jnp.min(jnp.where(scores == nmax, j_row, HALF), axis=1,
                   keepdims=True)
    nmax_ref[...] = nmax
    nidx_ref[...] = nidx


def _merge_kernel(nmax_col_ref, nmax_row_ref, nidx_ref, a_ref, b_ref,
                  unm_ref, dst_ref):
    rank = _rank_desc_col(
        nmax_col_ref[...], nmax_row_ref[...], HALF, 256)
    nidx = nidx_ref[...]
    a = a_ref[...]
    b = b_ref[...]
    m_row = jax.lax.broadcasted_iota(jnp.int32, (1, UNM), 1).astype(jnp.float32)
    unm_oh_t = jnp.where(rank - float(R) == m_row, 1.0, 0.0)
    unm_ref[...] = jax.lax.dot_general(
        unm_oh_t, a, (((0,), (0,)), ((), ())),
        preferred_element_type=jnp.float32, precision=jax.lax.Precision.HIGHEST)
    d_row = jax.lax.broadcasted_iota(jnp.int32, (1, HALF), 1)
    merge_t = jnp.where((nidx == d_row) & (rank < float(R)), 1.0, 0.0)
    accum = jax.lax.dot_general(
        merge_t, a, (((0,), (0,)), ((), ())),
        preferred_element_type=jnp.float32, precision=jax.lax.Precision.HIGHEST)
    ones_col = jnp.ones((HALF, 1), dtype=jnp.float32)
    counts = jax.lax.dot_general(
        merge_t, ones_col, (((0,), (0,)), ((), ())),
        preferred_element_type=jnp.float32, precision=jax.lax.Precision.HIGHEST) + 1.0
    dst_ref[...] = (b + accum) / counts


def kernel(hidden_states, self_attention_scores, Wq, Wk, Wv, Wo):
    scores = self_attention_scores.reshape(H, L, L)
    imp_tc = pl.pallas_call(
        _importance_kernel,
        grid=(L // _ROWS, H - _HSC),
        in_specs=[pl.BlockSpec((1, _ROWS, L), lambda c, h: (h + _HSC, c, 0))],
        out_specs=pl.BlockSpec((1, L), lambda c, h: (0, 0)),
        out_shape=jax.ShapeDtypeStruct((1, L), jnp.float32),
    )(scores)
    partials = _sc_colsum(self_attention_scores.reshape(H * L, L))
    imp = imp_tc + jnp.sum(partials, axis=0, keepdims=True)

    hs = hidden_states.reshape(L, D)
    new_tok = pl.pallas_call(
        _select_mha_kernel,
        out_shape=jax.ShapeDtypeStruct((1, D), jnp.float32),
    )(imp, imp.reshape(L, 1), hs, Wq, Wk, Wv, Wo)

    a_t = hs[0::2]
    b_t = hs[1::2]
    an_t = a_t / jnp.linalg.norm(a_t, axis=-1, keepdims=True)
    bn_t = b_t / jnp.linalg.norm(b_t, axis=-1, keepdims=True)
    nmax, nidx = pl.pallas_call(
        _node_stats_kernel,
        grid=(HALF // _BCH,),
        in_specs=[
            pl.BlockSpec((_BCH, D), lambda c: (c, 0)),
            pl.BlockSpec((HALF, D), lambda c: (0, 0)),
        ],
        out_specs=[
            pl.BlockSpec((_BCH, 1), lambda c: (c, 0)),
            pl.BlockSpec((_BCH, 1), lambda c: (c, 0)),
        ],
        out_shape=[
            jax.ShapeDtypeStruct((HALF, 1), jnp.float32),
            jax.ShapeDtypeStruct((HALF, 1), jnp.int32),
        ],
    )(an_t, bn_t)

    unm, dst_m = pl.pallas_call(
        _merge_kernel,
        out_shape=[
            jax.ShapeDtypeStruct((UNM, D), jnp.float32),
            jax.ShapeDtypeStruct((HALF, D), jnp.float32),
        ],
    )(nmax, nmax.reshape(1, HALF), nidx, a_t, b_t)

    cls = hs[0:1, :]
    return jnp.concatenate(
        [cls[None], unm[None], dst_m[None], new_tok[None]], axis=1)
